# baseline (speedup 1.0000x reference)
import functools

import jax
import jax.numpy as jnp
from jax import lax
from jax.experimental import pallas as pl
from jax.experimental.pallas import tpu as pltpu
from jax.experimental.pallas import tpu_sc as plsc

_NC = 2
_NS = 16
_CHUNK = 128
_BLK = 16
_G = 128


def _make_sc_agg(n, feat, n_pad, nblk, rpt):
    mesh = plsc.VectorSubcoreMesh(core_axis_name="c", subcore_axis_name="s",
                                  num_cores=1)

    @functools.partial(
        pl.kernel,
        mesh=mesh,
        out_type=jax.ShapeDtypeStruct((n_pad, feat), jnp.float32),
        scratch_types=[
            pltpu.VMEM((2, _BLK, _CHUNK), jnp.int32),
            pltpu.VMEM((2, _BLK, _CHUNK), jnp.int32),
            pltpu.VMEM((_CHUNK, feat), jnp.float32),
            pltpu.VMEM((_CHUNK, feat), jnp.float32),
            pltpu.VMEM_SHARED((n_pad, feat), jnp.float32),
            pltpu.SemaphoreType.DMA,
            pltpu.SemaphoreType.DMA,
            pltpu.SemaphoreType.DMA,
            pltpu.SemaphoreType.DMA,
        ],
    )
    def sc_agg(h_hbm, src_hbm, dst_hbm, z_hbm, out_hbm,
               src_v, dst_v, rows0, rows1, acc, sem0, sem1, semi0, semi1):
        sid = lax.axis_index("s")
        pltpu.sync_copy(z_hbm, acc.at[pl.ds(sid * rpt, rpt)])
        base = sid * nblk * _BLK
        semi = (semi0, semi1)
        pltpu.async_copy(src_hbm.at[pl.ds(base, _BLK)], src_v.at[0], semi0)
        pltpu.async_copy(dst_hbm.at[pl.ds(base, _BLK)], dst_v.at[0], semi0)
        plsc.subcore_barrier()

        for b in range(nblk):
            pb = b % 2
            sblk = src_v.at[pb]
            dblk = dst_v.at[pb]
            pltpu.make_async_copy(src_hbm.at[pl.ds(base, _BLK)], sblk,
                                  semi[pb]).wait()
            pltpu.make_async_copy(dst_hbm.at[pl.ds(base, _BLK)], dblk,
                                  semi[pb]).wait()
            if b + 1 < nblk:
                npb = (b + 1) % 2
                off = base + (b + 1) * _BLK
                pltpu.async_copy(src_hbm.at[pl.ds(off, _BLK)],
                                 src_v.at[npb], semi[npb])
                pltpu.async_copy(dst_hbm.at[pl.ds(off, _BLK)],
                                 dst_v.at[npb], semi[npb])

            pltpu.async_copy(h_hbm.at[sblk.at[0]], rows0, sem0)

            def body(i, carry, sblk=sblk, dblk=dblk):
                j = 2 * i
                pltpu.async_copy(h_hbm.at[sblk.at[j + 1]], rows1, sem1)
                pltpu.make_async_copy(h_hbm.at[sblk.at[j]], rows0,
                                      sem0).wait()
                pltpu.sync_copy(rows0, acc.at[dblk.at[j]], add=True)

                @pl.when(j + 2 < _BLK)
                def _():
                    pltpu.async_copy(h_hbm.at[sblk.at[j + 2]], rows0, sem0)

                pltpu.make_async_copy(h_hbm.at[sblk.at[j + 1]], rows1,
                                      sem1).wait()
                pltpu.sync_copy(rows1, acc.at[dblk.at[j + 1]], add=True)
                return carry

            lax.fori_loop(0, _BLK // 2, body, 0)
        plsc.subcore_barrier()
        pltpu.sync_copy(acc.at[pl.ds(sid * rpt, rpt)],
                        out_hbm.at[pl.ds(sid * rpt, rpt)])

    return sc_agg


def _dot(a, b):
    return jnp.dot(a, b, preferred_element_type=jnp.float32,
                   precision=lax.Precision.HIGHEST)


def _tc_layer(h, agg3, batch_row, p, n, n_pad):
    hdim = p["Wa"].shape[1]

    def body(h_ref, agg_ref, b_ref, wa_ref, ba_ref, ga_ref, bea_ref,
             wb_ref, bb_ref, gb_ref, beb_ref, z_ref, pool_ref):
        hn = h_ref[...] + agg_ref[:n, :]
        y = _dot(hn, wa_ref[...]) + ba_ref[...]
        m = jnp.mean(y, axis=0, keepdims=True)
        yc = y - m
        v = jnp.mean(yc * yc, axis=0, keepdims=True)
        y = jnp.maximum(yc * (ga_ref[...] * lax.rsqrt(v + 1e-5)) + bea_ref[...],
                        0.0)
        z = _dot(y, wb_ref[...]) + bb_ref[...]
        m2 = jnp.mean(z, axis=0, keepdims=True)
        zc = z - m2
        v2 = jnp.mean(zc * zc, axis=0, keepdims=True)
        z = jnp.maximum(zc * (gb_ref[...] * lax.rsqrt(v2 + 1e-5)) + beb_ref[...],
                        0.0)
        z_ref[...] = z
        oh_t = (lax.broadcasted_iota(jnp.int32, (_G, n), 0)
                == b_ref[...]).astype(jnp.float32)
        pool_ref[...] = _dot(oh_t, z)

    return pl.pallas_call(
        body,
        out_shape=(jax.ShapeDtypeStruct((n, hdim), jnp.float32),
                   jax.ShapeDtypeStruct((_G, hdim), jnp.float32)),
    )(h, agg3, batch_row,
      p["Wa"], p["ba"].reshape(1, -1), p["ga"].reshape(1, -1),
      p["bea"].reshape(1, -1),
      p["Wb"], p["bb"].reshape(1, -1), p["gb"].reshape(1, -1),
      p["beb"].reshape(1, -1))


def _head(pooled, w1, b1, w2, b2):
    nl = pooled.shape[0]
    c = w2.shape[1]

    def body(p_ref, w1_ref, b1_ref, w2_ref, b2_ref, o_ref):
        s = _dot(p_ref[0], w1_ref[0])
        for l in range(1, nl):
            s = s + _dot(p_ref[l], w1_ref[l])
        s = jnp.maximum(s + b1_ref[...], 0.0)
        o_ref[...] = _dot(s, w2_ref[...]) + b2_ref[...]

    return pl.pallas_call(
        body,
        out_shape=jax.ShapeDtypeStruct((_G, c), jnp.float32),
    )(pooled, w1, b1.reshape(1, -1), w2, b2.reshape(1, -1))


def kernel(x, edge_index, batch, params):
    n, feat = x.shape
    e = edge_index.shape[1]
    n_pad = ((n // 128) + 1) * 128
    rpt = n_pad // _NS
    per_core_blk = _NS * _BLK * _CHUNK
    nblk = -(-e // per_core_blk)
    e_pad = per_core_blk * nblk

    src = jnp.concatenate(
        [edge_index[0], jnp.zeros((e_pad - e,), jnp.int32)]).reshape(-1, _CHUNK)
    dst = jnp.concatenate(
        [edge_index[1], jnp.full((e_pad - e,), n, jnp.int32)]).reshape(-1, _CHUNK)
    zeros_rt = jnp.zeros((rpt, feat), jnp.float32)
    batch_row = batch.reshape(1, n)

    sc_agg = _make_sc_agg(n, feat, n_pad, nblk, rpt)

    h = x
    pools = []
    for p in params["convs"]:
        agg = sc_agg(h, src, dst, zeros_rt)
        h, pool = _tc_layer(h, agg, batch_row, p, n, n_pad)
        pools.append(pool)

    pooled = jnp.stack(pools)
    nl = len(pools)
    hdim = pooled.shape[2]
    w1 = params["lin1_W"].reshape(nl, hdim, -1)
    return _head(pooled, w1, params["lin1_b"], params["lin2_W"],
                 params["lin2_b"])


# speedup vs baseline: 3.4860x; 3.4860x over previous
import functools

import jax
import jax.numpy as jnp
from jax import lax
from jax.experimental import pallas as pl
from jax.experimental.pallas import tpu as pltpu
from jax.experimental.pallas import tpu_sc as plsc

_NC = 2
_NS = 16
_CHUNK = 128
_BLK = 16
_G = 128


def _make_sc_agg(n, feat, n_pad, nblk, rpt):
    mesh = plsc.VectorSubcoreMesh(core_axis_name="c", subcore_axis_name="s")

    @functools.partial(
        pl.kernel,
        mesh=mesh,
        out_type=jax.ShapeDtypeStruct((_NC * n_pad, feat), jnp.float32),
        scratch_types=[
            pltpu.VMEM((2, _BLK, _CHUNK), jnp.int32),
            pltpu.VMEM((2, _BLK, _CHUNK), jnp.int32),
            pltpu.VMEM((_CHUNK, feat), jnp.float32),
            pltpu.VMEM((_CHUNK, feat), jnp.float32),
            pltpu.VMEM_SHARED((n_pad, feat), jnp.float32),
            pltpu.SemaphoreType.DMA,
            pltpu.SemaphoreType.DMA,
            pltpu.SemaphoreType.DMA,
            pltpu.SemaphoreType.DMA,
        ],
    )
    def sc_agg(h_hbm, src_hbm, dst_hbm, z_hbm, out_hbm,
               src_v, dst_v, rows0, rows1, acc, sem0, sem1, semi0, semi1):
        cid = lax.axis_index("c")
        sid = lax.axis_index("s")
        pltpu.sync_copy(z_hbm, acc.at[pl.ds(sid * rpt, rpt)])
        base = (cid * _NS + sid) * nblk * _BLK
        semi = (semi0, semi1)
        pltpu.async_copy(src_hbm.at[pl.ds(base, _BLK)], src_v.at[0], semi0)
        pltpu.async_copy(dst_hbm.at[pl.ds(base, _BLK)], dst_v.at[0], semi0)
        plsc.subcore_barrier()

        for b in range(nblk):
            pb = b % 2
            sblk = src_v.at[pb]
            dblk = dst_v.at[pb]
            pltpu.make_async_copy(src_hbm.at[pl.ds(base, _BLK)], sblk,
                                  semi[pb]).wait()
            pltpu.make_async_copy(dst_hbm.at[pl.ds(base, _BLK)], dblk,
                                  semi[pb]).wait()
            if b + 1 < nblk:
                npb = (b + 1) % 2
                off = base + (b + 1) * _BLK
                pltpu.async_copy(src_hbm.at[pl.ds(off, _BLK)],
                                 src_v.at[npb], semi[npb])
                pltpu.async_copy(dst_hbm.at[pl.ds(off, _BLK)],
                                 dst_v.at[npb], semi[npb])

            pltpu.async_copy(h_hbm.at[sblk.at[0]], rows0, sem0)

            def body(i, carry, sblk=sblk, dblk=dblk):
                j = 2 * i
                pltpu.async_copy(h_hbm.at[sblk.at[j + 1]], rows1, sem1)
                pltpu.make_async_copy(h_hbm.at[sblk.at[j]], rows0,
                                      sem0).wait()
                pltpu.sync_copy(rows0, acc.at[dblk.at[j]], add=True)

                @pl.when(j + 2 < _BLK)
                def _():
                    pltpu.async_copy(h_hbm.at[sblk.at[j + 2]], rows0, sem0)

                pltpu.make_async_copy(h_hbm.at[sblk.at[j + 1]], rows1,
                                      sem1).wait()
                pltpu.sync_copy(rows1, acc.at[dblk.at[j + 1]], add=True)
                return carry

            lax.fori_loop(0, _BLK // 2, body, 0)
        plsc.subcore_barrier()
        pltpu.sync_copy(acc.at[pl.ds(sid * rpt, rpt)],
                        out_hbm.at[pl.ds(cid * n_pad + sid * rpt, rpt)])

    return sc_agg


def _dot(a, b):
    return jnp.dot(a, b, preferred_element_type=jnp.float32,
                   precision=lax.Precision.HIGHEST)


def _tc_layer(h, agg3, batch_row, p, n, n_pad):
    hdim = p["Wa"].shape[1]

    def body(h_ref, agg_ref, b_ref, wa_ref, ba_ref, ga_ref, bea_ref,
             wb_ref, bb_ref, gb_ref, beb_ref, z_ref, pool_ref):
        hn = h_ref[...] + agg_ref[0, :n, :] + agg_ref[1, :n, :]
        y = _dot(hn, wa_ref[...]) + ba_ref[...]
        m = jnp.mean(y, axis=0, keepdims=True)
        yc = y - m
        v = jnp.mean(yc * yc, axis=0, keepdims=True)
        y = jnp.maximum(yc * (ga_ref[...] * lax.rsqrt(v + 1e-5)) + bea_ref[...],
                        0.0)
        z = _dot(y, wb_ref[...]) + bb_ref[...]
        m2 = jnp.mean(z, axis=0, keepdims=True)
        zc = z - m2
        v2 = jnp.mean(zc * zc, axis=0, keepdims=True)
        z = jnp.maximum(zc * (gb_ref[...] * lax.rsqrt(v2 + 1e-5)) + beb_ref[...],
                        0.0)
        z_ref[...] = z
        oh_t = (lax.broadcasted_iota(jnp.int32, (_G, n), 0)
                == b_ref[...]).astype(jnp.float32)
        pool_ref[...] = _dot(oh_t, z)

    return pl.pallas_call(
        body,
        out_shape=(jax.ShapeDtypeStruct((n, hdim), jnp.float32),
                   jax.ShapeDtypeStruct((_G, hdim), jnp.float32)),
    )(h, agg3, batch_row,
      p["Wa"], p["ba"].reshape(1, -1), p["ga"].reshape(1, -1),
      p["bea"].reshape(1, -1),
      p["Wb"], p["bb"].reshape(1, -1), p["gb"].reshape(1, -1),
      p["beb"].reshape(1, -1))


def _head(pooled, w1, b1, w2, b2):
    nl = pooled.shape[0]
    c = w2.shape[1]

    def body(p_ref, w1_ref, b1_ref, w2_ref, b2_ref, o_ref):
        s = _dot(p_ref[0], w1_ref[0])
        for l in range(1, nl):
            s = s + _dot(p_ref[l], w1_ref[l])
        s = jnp.maximum(s + b1_ref[...], 0.0)
        o_ref[...] = _dot(s, w2_ref[...]) + b2_ref[...]

    return pl.pallas_call(
        body,
        out_shape=jax.ShapeDtypeStruct((_G, c), jnp.float32),
    )(pooled, w1, b1.reshape(1, -1), w2, b2.reshape(1, -1))


def kernel(x, edge_index, batch, params):
    n, feat = x.shape
    e = edge_index.shape[1]
    n_pad = ((n // 128) + 1) * 128
    rpt = n_pad // _NS
    per_worker_blk = _NC * _NS * _BLK * _CHUNK
    nblk = -(-e // per_worker_blk)
    e_pad = per_worker_blk * nblk

    npad_e = e_pad - e
    pad_i = jnp.arange(npad_e, dtype=jnp.int32)
    src = jnp.concatenate(
        [edge_index[0], pad_i % n]).reshape(-1, _CHUNK)
    dst = jnp.concatenate(
        [edge_index[1], n + pad_i % (n_pad - n)]).reshape(-1, _CHUNK)
    zeros_rt = jnp.zeros((rpt, feat), jnp.float32)
    batch_row = batch.reshape(1, n)

    sc_agg = _make_sc_agg(n, feat, n_pad, nblk, rpt)

    h = x
    pools = []
    for p in params["convs"]:
        agg = sc_agg(h, src, dst, zeros_rt)
        agg3 = agg.reshape(_NC, n_pad, feat)
        h, pool = _tc_layer(h, agg3, batch_row, p, n, n_pad)
        pools.append(pool)

    pooled = jnp.stack(pools)
    nl = len(pools)
    hdim = pooled.shape[2]
    w1 = params["lin1_W"].reshape(nl, hdim, -1)
    return _head(pooled, w1, params["lin1_b"], params["lin2_W"],
                 params["lin2_b"])


# speedup vs baseline: 3.5353x; 1.0141x over previous
import functools

import jax
import jax.numpy as jnp
from jax import lax
from jax.experimental import pallas as pl
from jax.experimental.pallas import tpu as pltpu
from jax.experimental.pallas import tpu_sc as plsc

_NC = 2
_NS = 16
_CHUNK = 128
_BLK = 16
_G = 128


def _make_sc_agg(n, feat, n_pad, nblk, rpt):
    mesh = plsc.VectorSubcoreMesh(core_axis_name="c", subcore_axis_name="s")

    @functools.partial(
        pl.kernel,
        mesh=mesh,
        out_type=jax.ShapeDtypeStruct((_NC * n_pad, feat), jnp.float32),
        scratch_types=[
            pltpu.VMEM((2, _BLK, _CHUNK), jnp.int32),
            pltpu.VMEM((2, _BLK, _CHUNK), jnp.int32),
            pltpu.VMEM((_CHUNK, feat), jnp.float32),
            pltpu.VMEM((_CHUNK, feat), jnp.float32),
            pltpu.VMEM_SHARED((n_pad, feat), jnp.float32),
            pltpu.SemaphoreType.DMA,
            pltpu.SemaphoreType.DMA,
            pltpu.SemaphoreType.DMA,
            pltpu.SemaphoreType.DMA,
        ],
    )
    def sc_agg(h_hbm, src_hbm, dst_hbm, z_hbm, out_hbm,
               src_v, dst_v, rows0, rows1, acc, sem0, sem1, semi0, semi1):
        cid = lax.axis_index("c")
        sid = lax.axis_index("s")
        pltpu.sync_copy(z_hbm, acc.at[pl.ds(sid * rpt, rpt)])
        base = (cid * _NS + sid) * nblk * _BLK
        semi = (semi0, semi1)
        pltpu.async_copy(src_hbm.at[pl.ds(base, _BLK)], src_v.at[0], semi0)
        pltpu.async_copy(dst_hbm.at[pl.ds(base, _BLK)], dst_v.at[0], semi0)
        plsc.subcore_barrier()

        for b in range(nblk):
            pb = b % 2
            sblk = src_v.at[pb]
            dblk = dst_v.at[pb]
            pltpu.make_async_copy(src_hbm.at[pl.ds(base, _BLK)], sblk,
                                  semi[pb]).wait()
            pltpu.make_async_copy(dst_hbm.at[pl.ds(base, _BLK)], dblk,
                                  semi[pb]).wait()
            if b + 1 < nblk:
                npb = (b + 1) % 2
                off = base + (b + 1) * _BLK
                pltpu.async_copy(src_hbm.at[pl.ds(off, _BLK)],
                                 src_v.at[npb], semi[npb])
                pltpu.async_copy(dst_hbm.at[pl.ds(off, _BLK)],
                                 dst_v.at[npb], semi[npb])

            pltpu.async_copy(h_hbm.at[sblk.at[0]], rows0, sem0)

            def body(i, carry, sblk=sblk, dblk=dblk):
                j = 2 * i
                pltpu.async_copy(h_hbm.at[sblk.at[j + 1]], rows1, sem1)
                pltpu.make_async_copy(h_hbm.at[sblk.at[j]], rows0,
                                      sem0).wait()
                pltpu.sync_copy(rows0, acc.at[dblk.at[j]], add=True)

                @pl.when(j + 2 < _BLK)
                def _():
                    pltpu.async_copy(h_hbm.at[sblk.at[j + 2]], rows0, sem0)

                pltpu.make_async_copy(h_hbm.at[sblk.at[j + 1]], rows1,
                                      sem1).wait()
                pltpu.sync_copy(rows1, acc.at[dblk.at[j + 1]], add=True)
                return carry

            lax.fori_loop(0, _BLK // 2, body, 0)
        plsc.subcore_barrier()
        pltpu.sync_copy(acc.at[pl.ds(sid * rpt, rpt)],
                        out_hbm.at[pl.ds(cid * n_pad + sid * rpt, rpt)])

    return sc_agg


def _dot(a, b):
    return jnp.dot(a, b, preferred_element_type=jnp.float32,
                   precision=lax.Precision.HIGHEST)


def _tc_layer(h, agg3, p, n, n_pad):
    hdim = p["Wa"].shape[1]

    def body(h_ref, agg_ref, wa_ref, ba_ref, ga_ref, bea_ref,
             wb_ref, bb_ref, gb_ref, beb_ref, z_ref):
        hn = h_ref[...] + agg_ref[0, :n, :] + agg_ref[1, :n, :]
        y = _dot(hn, wa_ref[...]) + ba_ref[...]
        m = jnp.mean(y, axis=0, keepdims=True)
        yc = y - m
        v = jnp.mean(yc * yc, axis=0, keepdims=True)
        y = jnp.maximum(yc * (ga_ref[...] * lax.rsqrt(v + 1e-5)) + bea_ref[...],
                        0.0)
        z = _dot(y, wb_ref[...]) + bb_ref[...]
        m2 = jnp.mean(z, axis=0, keepdims=True)
        zc = z - m2
        v2 = jnp.mean(zc * zc, axis=0, keepdims=True)
        z = jnp.maximum(zc * (gb_ref[...] * lax.rsqrt(v2 + 1e-5)) + beb_ref[...],
                        0.0)
        z_ref[...] = z

    return pl.pallas_call(
        body,
        out_shape=jax.ShapeDtypeStruct((n, hdim), jnp.float32),
    )(h, agg3,
      p["Wa"], p["ba"].reshape(1, -1), p["ga"].reshape(1, -1),
      p["bea"].reshape(1, -1),
      p["Wb"], p["bb"].reshape(1, -1), p["gb"].reshape(1, -1),
      p["beb"].reshape(1, -1))


def _tc_pool(z, batch_row, n):
    hdim = z.shape[1]

    def body(z_ref, b_ref, pool_ref):
        oh_t = (lax.broadcasted_iota(jnp.int32, (_G, n), 0)
                == b_ref[...]).astype(jnp.float32)
        pool_ref[...] = jnp.dot(oh_t, z_ref[...],
                                preferred_element_type=jnp.float32)

    return pl.pallas_call(
        body,
        out_shape=jax.ShapeDtypeStruct((_G, hdim), jnp.float32),
    )(z, batch_row)


def _head(pooled, w1, b1, w2, b2):
    nl = pooled.shape[0]
    c = w2.shape[1]

    def body(p_ref, w1_ref, b1_ref, w2_ref, b2_ref, o_ref):
        dot = lambda a, b: jnp.dot(a, b, preferred_element_type=jnp.float32)
        s = dot(p_ref[0], w1_ref[0])
        for l in range(1, nl):
            s = s + dot(p_ref[l], w1_ref[l])
        s = jnp.maximum(s + b1_ref[...], 0.0)
        o_ref[...] = dot(s, w2_ref[...]) + b2_ref[...]

    return pl.pallas_call(
        body,
        out_shape=jax.ShapeDtypeStruct((_G, c), jnp.float32),
    )(pooled, w1, b1.reshape(1, -1), w2, b2.reshape(1, -1))


def kernel(x, edge_index, batch, params):
    n, feat = x.shape
    e = edge_index.shape[1]
    n_pad = ((n // 128) + 1) * 128
    rpt = n_pad // _NS
    per_worker_blk = _NC * _NS * _BLK * _CHUNK
    nblk = -(-e // per_worker_blk)
    e_pad = per_worker_blk * nblk

    npad_e = e_pad - e
    pad_i = jnp.arange(npad_e, dtype=jnp.int32)
    src = jnp.concatenate(
        [edge_index[0], pad_i % n]).reshape(-1, _CHUNK)
    dst = jnp.concatenate(
        [edge_index[1], n + pad_i % (n_pad - n)]).reshape(-1, _CHUNK)
    zeros_rt = jnp.zeros((rpt, feat), jnp.float32)
    batch_row = batch.reshape(1, n)

    sc_agg = _make_sc_agg(n, feat, n_pad, nblk, rpt)

    h = x
    pools = []
    for p in params["convs"]:
        agg = sc_agg(h, src, dst, zeros_rt)
        agg3 = agg.reshape(_NC, n_pad, feat)
        h = _tc_layer(h, agg3, p, n, n_pad)
        pools.append(_tc_pool(h, batch_row, n))

    pooled = jnp.stack(pools)
    nl = len(pools)
    hdim = pooled.shape[2]
    w1 = params["lin1_W"].reshape(nl, hdim, -1)
    return _head(pooled, w1, params["lin1_b"], params["lin2_W"],
                 params["lin2_b"])


# speedup vs baseline: 3.9188x; 1.1085x over previous
import functools

import jax
import jax.numpy as jnp
import numpy as np
from jax import lax
from jax.experimental import pallas as pl
from jax.experimental.pallas import tpu as pltpu
from jax.experimental.pallas import tpu_sc as plsc

_NC = 2
_NS = 16
_CHUNK = 128
_BLK = 16
_G = 128


def _make_sc_agg(n, feat, n_pad, nblk, rpt):
    mesh = plsc.VectorSubcoreMesh(core_axis_name="c", subcore_axis_name="s")

    @functools.partial(
        pl.kernel,
        mesh=mesh,
        out_type=jax.ShapeDtypeStruct((_NC * n_pad, feat), jnp.float32),
        scratch_types=[
            pltpu.VMEM((2, _BLK, _CHUNK), jnp.int32),
            pltpu.VMEM((2, _BLK, _CHUNK), jnp.int32),
            pltpu.VMEM((_CHUNK, feat), jnp.float32),
            pltpu.VMEM((_CHUNK, feat), jnp.float32),
            pltpu.VMEM_SHARED((n_pad, feat), jnp.float32),
            pltpu.SemaphoreType.DMA,
            pltpu.SemaphoreType.DMA,
            pltpu.SemaphoreType.DMA,
            pltpu.SemaphoreType.DMA,
        ],
    )
    def sc_agg(h_hbm, src_hbm, dst_hbm, z_hbm, out_hbm,
               src_v, dst_v, rows0, rows1, acc, sem0, sem1, semi0, semi1):
        cid = lax.axis_index("c")
        sid = lax.axis_index("s")
        pltpu.sync_copy(z_hbm, acc.at[pl.ds(sid * rpt, rpt)])
        base = (cid * _NS + sid) * nblk * _BLK
        semi = (semi0, semi1)
        pltpu.async_copy(src_hbm.at[pl.ds(base, _BLK)], src_v.at[0], semi0)
        pltpu.async_copy(dst_hbm.at[pl.ds(base, _BLK)], dst_v.at[0], semi0)
        plsc.subcore_barrier()

        for b in range(nblk):
            pb = b % 2
            sblk = src_v.at[pb]
            dblk = dst_v.at[pb]
            pltpu.make_async_copy(src_hbm.at[pl.ds(base, _BLK)], sblk,
                                  semi[pb]).wait()
            pltpu.make_async_copy(dst_hbm.at[pl.ds(base, _BLK)], dblk,
                                  semi[pb]).wait()
            if b + 1 < nblk:
                npb = (b + 1) % 2
                off = base + (b + 1) * _BLK
                pltpu.async_copy(src_hbm.at[pl.ds(off, _BLK)],
                                 src_v.at[npb], semi[npb])
                pltpu.async_copy(dst_hbm.at[pl.ds(off, _BLK)],
                                 dst_v.at[npb], semi[npb])

            pltpu.async_copy(h_hbm.at[sblk.at[0]], rows0, sem0)

            def body(i, carry, sblk=sblk, dblk=dblk):
                j = 2 * i
                pltpu.async_copy(h_hbm.at[sblk.at[j + 1]], rows1, sem1)
                pltpu.make_async_copy(h_hbm.at[sblk.at[j]], rows0,
                                      sem0).wait()
                pltpu.sync_copy(rows0, acc.at[dblk.at[j]], add=True)

                @pl.when(j + 2 < _BLK)
                def _():
                    pltpu.async_copy(h_hbm.at[sblk.at[j + 2]], rows0, sem0)

                pltpu.make_async_copy(h_hbm.at[sblk.at[j + 1]], rows1,
                                      sem1).wait()
                pltpu.sync_copy(rows1, acc.at[dblk.at[j + 1]], add=True)
                return carry

            lax.fori_loop(0, _BLK // 2, body, 0)
        plsc.subcore_barrier()
        pltpu.sync_copy(acc.at[pl.ds(sid * rpt, rpt)],
                        out_hbm.at[pl.ds(cid * n_pad + sid * rpt, rpt)])

    return sc_agg


def _dot(a, b):
    return jnp.dot(a, b, preferred_element_type=jnp.float32)


def _tc_layer(h, agg3, p, n, n_pad):
    hdim = p["Wa"].shape[1]

    def body(h_ref, agg_ref, wa_ref, ba_ref, ga_ref, bea_ref,
             wb_ref, bb_ref, gb_ref, beb_ref, z_ref):
        hn = h_ref[...] + agg_ref[0, :n, :] + agg_ref[1, :n, :]
        y = _dot(hn, wa_ref[...]) + ba_ref[...]
        m = jnp.mean(y, axis=0, keepdims=True)
        yc = y - m
        v = jnp.mean(yc * yc, axis=0, keepdims=True)
        y = jnp.maximum(yc * (ga_ref[...] * lax.rsqrt(v + 1e-5)) + bea_ref[...],
                        0.0)
        z = _dot(y, wb_ref[...]) + bb_ref[...]
        m2 = jnp.mean(z, axis=0, keepdims=True)
        zc = z - m2
        v2 = jnp.mean(zc * zc, axis=0, keepdims=True)
        z = jnp.maximum(zc * (gb_ref[...] * lax.rsqrt(v2 + 1e-5)) + beb_ref[...],
                        0.0)
        z_ref[...] = z

    return pl.pallas_call(
        body,
        out_shape=jax.ShapeDtypeStruct((n, hdim), jnp.float32),
    )(h, agg3,
      p["Wa"], p["ba"].reshape(1, -1), p["ga"].reshape(1, -1),
      p["bea"].reshape(1, -1),
      p["Wb"], p["bb"].reshape(1, -1), p["gb"].reshape(1, -1),
      p["beb"].reshape(1, -1))


def _tc_pool(z, batch_row, n):
    hdim = z.shape[1]

    def body(z_ref, b_ref, pool_ref):
        oh_t = (lax.broadcasted_iota(jnp.int32, (_G, n), 0)
                == b_ref[...]).astype(jnp.float32)
        pool_ref[...] = jnp.dot(oh_t, z_ref[...],
                                preferred_element_type=jnp.float32)

    return pl.pallas_call(
        body,
        out_shape=jax.ShapeDtypeStruct((_G, hdim), jnp.float32),
    )(z, batch_row)


def _head(pooled3, z_last, batch_row, w1, b1, w2, b2, n):
    nl = w1.shape[0]
    c = w2.shape[1]

    def body(p_ref, z_ref, b_ref, w1_ref, b1_ref, w2_ref, b2_ref, o_ref):
        oh_t = (lax.broadcasted_iota(jnp.int32, (_G, n), 0)
                == b_ref[...]).astype(jnp.float32)
        s = _dot(oh_t @ z_ref[...], w1_ref[nl - 1])
        for l in range(nl - 1):
            s = s + _dot(p_ref[l], w1_ref[l])
        s = jnp.maximum(s + b1_ref[...], 0.0)
        o_ref[...] = _dot(s, w2_ref[...]) + b2_ref[...]

    return pl.pallas_call(
        body,
        out_shape=jax.ShapeDtypeStruct((_G, c), jnp.float32),
    )(pooled3, z_last, batch_row, w1, b1.reshape(1, -1), w2,
      b2.reshape(1, -1))


def kernel(x, edge_index, batch, params):
    n, feat = x.shape
    e = edge_index.shape[1]
    n_pad = ((n // 128) + 1) * 128
    rpt = n_pad // _NS
    per_worker_blk = _NC * _NS * _BLK * _CHUNK
    nblk = -(-e // per_worker_blk)
    e_pad = per_worker_blk * nblk

    npad_e = e_pad - e
    pad_i = np.arange(npad_e, dtype=np.int32)
    pad_src = jnp.asarray(pad_i % n)
    pad_dst = jnp.asarray(n + pad_i % (n_pad - n))
    src = jnp.concatenate([edge_index[0], pad_src]).reshape(-1, _CHUNK)
    dst = jnp.concatenate([edge_index[1], pad_dst]).reshape(-1, _CHUNK)
    zeros_rt = jnp.zeros((rpt, feat), jnp.float32)
    batch_row = batch.reshape(1, n)

    sc_agg = _make_sc_agg(n, feat, n_pad, nblk, rpt)

    h = x
    pools = []
    convs = params["convs"]
    for p in convs:
        agg = sc_agg(h, src, dst, zeros_rt)
        agg3 = agg.reshape(_NC, n_pad, feat)
        h = _tc_layer(h, agg3, p, n, n_pad)
        if len(pools) < len(convs) - 1:
            pools.append(_tc_pool(h, batch_row, n))

    pooled3 = jnp.stack(pools)
    nl = len(convs)
    hdim = h.shape[1]
    w1 = params["lin1_W"].reshape(nl, hdim, -1)
    return _head(pooled3, h, batch_row, w1, params["lin1_b"],
                 params["lin2_W"], params["lin2_b"], n)


# speedup vs baseline: 4.0667x; 1.0378x over previous
import functools

import jax
import jax.numpy as jnp
import numpy as np
from jax import lax
from jax.experimental import pallas as pl
from jax.experimental.pallas import tpu as pltpu
from jax.experimental.pallas import tpu_sc as plsc

_NC = 2
_NS = 16
_CHUNK = 128
_BLK = 16
_G = 128


def _make_sc_agg(n, feat, n_pad, nblk, rpt):
    mesh = plsc.VectorSubcoreMesh(core_axis_name="c", subcore_axis_name="s")

    @functools.partial(
        pl.kernel,
        mesh=mesh,
        out_type=jax.ShapeDtypeStruct((_NC * n_pad, feat), jnp.float32),
        scratch_types=[
            pltpu.VMEM((2, _BLK, _CHUNK), jnp.int32),
            pltpu.VMEM((2, _BLK, _CHUNK), jnp.int32),
            pltpu.VMEM((_CHUNK, feat), jnp.float32),
            pltpu.VMEM((_CHUNK, feat), jnp.float32),
            pltpu.VMEM((64, feat), jnp.float32),
            pltpu.VMEM_SHARED((n_pad, feat), jnp.float32),
            pltpu.SemaphoreType.DMA,
            pltpu.SemaphoreType.DMA,
            pltpu.SemaphoreType.DMA,
            pltpu.SemaphoreType.DMA,
        ],
    )
    def sc_agg(h_hbm, src_hbm, dst_hbm, out_hbm,
               src_v, dst_v, rows0, rows1, zbuf, acc,
               sem0, sem1, semi0, semi1):
        cid = lax.axis_index("c")
        sid = lax.axis_index("s")
        zv = jnp.zeros((16,), jnp.float32)

        def zrow(r, carry):
            for k in range(feat // 16):
                zbuf[r, pl.ds(k * 16, 16)] = zv
            return carry

        lax.fori_loop(0, 64, zrow, 0)
        zoff = 0
        while zoff < rpt:
            zn = min(64, rpt - zoff)
            pltpu.sync_copy(zbuf.at[pl.ds(0, zn)],
                            acc.at[pl.ds(sid * rpt + zoff, zn)])
            zoff += zn
        base = (cid * _NS + sid) * nblk * _BLK
        semi = (semi0, semi1)
        pltpu.async_copy(src_hbm.at[pl.ds(base, _BLK)], src_v.at[0], semi0)
        pltpu.async_copy(dst_hbm.at[pl.ds(base, _BLK)], dst_v.at[0], semi0)
        plsc.subcore_barrier()

        for b in range(nblk):
            pb = b % 2
            sblk = src_v.at[pb]
            dblk = dst_v.at[pb]
            pltpu.make_async_copy(src_hbm.at[pl.ds(base, _BLK)], sblk,
                                  semi[pb]).wait()
            pltpu.make_async_copy(dst_hbm.at[pl.ds(base, _BLK)], dblk,
                                  semi[pb]).wait()
            if b + 1 < nblk:
                npb = (b + 1) % 2
                off = base + (b + 1) * _BLK
                pltpu.async_copy(src_hbm.at[pl.ds(off, _BLK)],
                                 src_v.at[npb], semi[npb])
                pltpu.async_copy(dst_hbm.at[pl.ds(off, _BLK)],
                                 dst_v.at[npb], semi[npb])

            pltpu.async_copy(h_hbm.at[sblk.at[0]], rows0, sem0)

            def body(i, carry, sblk=sblk, dblk=dblk):
                j = 2 * i
                pltpu.async_copy(h_hbm.at[sblk.at[j + 1]], rows1, sem1)
                pltpu.make_async_copy(h_hbm.at[sblk.at[j]], rows0,
                                      sem0).wait()
                pltpu.sync_copy(rows0, acc.at[dblk.at[j]], add=True)

                @pl.when(j + 2 < _BLK)
                def _():
                    pltpu.async_copy(h_hbm.at[sblk.at[j + 2]], rows0, sem0)

                pltpu.make_async_copy(h_hbm.at[sblk.at[j + 1]], rows1,
                                      sem1).wait()
                pltpu.sync_copy(rows1, acc.at[dblk.at[j + 1]], add=True)
                return carry

            lax.fori_loop(0, _BLK // 2, body, 0)
        plsc.subcore_barrier()
        pltpu.sync_copy(acc.at[pl.ds(sid * rpt, rpt)],
                        out_hbm.at[pl.ds(cid * n_pad + sid * rpt, rpt)])

    return sc_agg


def _dot(a, b):
    return jnp.dot(a, b, preferred_element_type=jnp.float32)


def _tc_layer(h, agg3, p, n, n_pad):
    hdim = p["Wa"].shape[1]

    def body(h_ref, agg_ref, wa_ref, ga_ref, bea_ref,
             wb_ref, gb_ref, beb_ref, z_ref):
        hn = h_ref[...] + agg_ref[0, :n, :] + agg_ref[1, :n, :]
        y = _dot(hn, wa_ref[...])
        m = jnp.mean(y, axis=0, keepdims=True)
        yc = y - m
        v = jnp.mean(yc * yc, axis=0, keepdims=True)
        y = jnp.maximum(yc * (ga_ref[...] * lax.rsqrt(v + 1e-5)) + bea_ref[...],
                        0.0)
        z = _dot(y, wb_ref[...])
        m2 = jnp.mean(z, axis=0, keepdims=True)
        zc = z - m2
        v2 = jnp.mean(zc * zc, axis=0, keepdims=True)
        z = jnp.maximum(zc * (gb_ref[...] * lax.rsqrt(v2 + 1e-5)) + beb_ref[...],
                        0.0)
        z_ref[...] = z

    return pl.pallas_call(
        body,
        out_shape=jax.ShapeDtypeStruct((n, hdim), jnp.float32),
    )(h, agg3,
      p["Wa"], p["ga"].reshape(1, -1), p["bea"].reshape(1, -1),
      p["Wb"], p["gb"].reshape(1, -1), p["beb"].reshape(1, -1))


def _tc_pool(z, batch_row, n):
    hdim = z.shape[1]

    def body(z_ref, b_ref, pool_ref):
        oh_t = (lax.broadcasted_iota(jnp.int32, (_G, n), 0)
                == b_ref[...]).astype(jnp.float32)
        pool_ref[...] = jnp.dot(oh_t, z_ref[...],
                                preferred_element_type=jnp.float32)

    return pl.pallas_call(
        body,
        out_shape=jax.ShapeDtypeStruct((_G, hdim), jnp.float32),
    )(z, batch_row)


def _head(pooled3, z_last, batch_row, w1, b1, w2, b2, n):
    nl = w1.shape[0]
    c = w2.shape[1]

    def body(p_ref, z_ref, b_ref, w1_ref, b1_ref, w2_ref, b2_ref, o_ref):
        oh_t = (lax.broadcasted_iota(jnp.int32, (_G, n), 0)
                == b_ref[...]).astype(jnp.float32)
        s = _dot(oh_t @ z_ref[...], w1_ref[nl - 1])
        for l in range(nl - 1):
            s = s + _dot(p_ref[l], w1_ref[l])
        s = jnp.maximum(s + b1_ref[...], 0.0)
        o_ref[...] = _dot(s, w2_ref[...]) + b2_ref[...]

    return pl.pallas_call(
        body,
        out_shape=jax.ShapeDtypeStruct((_G, c), jnp.float32),
    )(pooled3, z_last, batch_row, w1, b1.reshape(1, -1), w2,
      b2.reshape(1, -1))


def kernel(x, edge_index, batch, params):
    n, feat = x.shape
    e = edge_index.shape[1]
    n_pad = ((n // 128) + 1) * 128
    rpt = n_pad // _NS
    per_worker_blk = _NC * _NS * _BLK * _CHUNK
    nblk = -(-e // per_worker_blk)
    e_pad = per_worker_blk * nblk

    npad_e = e_pad - e
    pad_i = np.arange(npad_e, dtype=np.int32)
    pad_src = jnp.asarray(pad_i % n)
    pad_dst = jnp.asarray(n + pad_i % (n_pad - n))
    src = jnp.concatenate([edge_index[0], pad_src]).reshape(-1, _CHUNK)
    dst = jnp.concatenate([edge_index[1], pad_dst]).reshape(-1, _CHUNK)
    batch_row = batch.reshape(1, n)

    sc_agg = _make_sc_agg(n, feat, n_pad, nblk, rpt)

    h = x
    pools = []
    convs = params["convs"]
    for p in convs:
        agg = sc_agg(h, src, dst)
        agg3 = agg.reshape(_NC, n_pad, feat)
        h = _tc_layer(h, agg3, p, n, n_pad)
        if len(pools) < len(convs) - 1:
            pools.append(_tc_pool(h, batch_row, n))

    pooled3 = jnp.stack(pools)
    nl = len(convs)
    hdim = h.shape[1]
    w1 = params["lin1_W"].reshape(nl, hdim, -1)
    return _head(pooled3, h, batch_row, w1, params["lin1_b"],
                 params["lin2_W"], params["lin2_b"], n)
